# baseline (speedup 1.0000x reference)
import functools
import jax
import jax.numpy as jnp
from jax import lax
from jax.experimental import pallas as pl
from jax.experimental.pallas import tpu as pltpu
from jax.experimental.pallas import tpu_sc as plsc

B, C, H, W = 16, 4, 512, 512
P = B * C
NC, NS = 2, 16
NW = NC * NS
RS = H // NW
SE = RS * W
PE = H * W


def _tid(b, c):
    return jnp.where(
        b == 2,
        jnp.where(c == 2, 4, 3),
        jnp.where(c < 2, 0, jnp.where(c == 2, 1, 2)),
    )


def _tc_body(ix_ref, iy_ref, y_ref, out_ref, tmpl_ref):
    i = pl.program_id(0)

    @pl.when(i == 0)
    def _build_templates():
        h = jax.lax.broadcasted_iota(jnp.int32, (H, W), 0)
        inrows = (h >= 3) & (h < 11) & ((h % 2) == 1)
        tmpl_ref[0] = jnp.where(inrows, 3.0, 1.0).astype(jnp.float32)
        tmpl_ref[1] = jnp.zeros((H, W), jnp.float32)
        tmpl_ref[2] = jnp.zeros((H, W), jnp.float32)
        tmpl_ref[3] = jnp.full((H, W), 4.0, jnp.float32)
        tmpl_ref[4] = jnp.full((H, W), 4.0, jnp.float32)
        hh = jax.lax.broadcasted_iota(jnp.int32, (16, 128), 0)
        ww = jax.lax.broadcasted_iota(jnp.int32, (16, 128), 1)

        def upd(t, m):
            return jnp.where((hh == ix_ref[t]) & (ww == iy_ref[t]), 1.0, m)

        m = jax.lax.fori_loop(0, 64, upd, jnp.zeros((16, 128), jnp.float32))
        hit = m > 0.5
        tmpl_ref[1, 0:16, 0:128] = jnp.where(hit, 1.0, 0.0).astype(jnp.float32)
        tmpl_ref[4, 0:16, 0:128] = jnp.where(hit, 1.0, 4.0).astype(jnp.float32)

    out_ref[0] = y_ref[0] + tmpl_ref[_tid(i // C, i % C)]


def _tc_out(y_flat, index_x, index_y):
    smem = pl.BlockSpec(memory_space=pltpu.SMEM)
    return pl.pallas_call(
        _tc_body,
        grid=(P,),
        in_specs=[smem, smem, pl.BlockSpec((1, H, W), lambda i: (i, 0, 0))],
        out_specs=pl.BlockSpec((1, H, W), lambda i: (i, 0, 0)),
        out_shape=jax.ShapeDtypeStruct((P, H, W), jnp.float32),
        scratch_shapes=[pltpu.VMEM((5, H, W), jnp.float32)],
        compiler_params=pltpu.CompilerParams(
            dimension_semantics=("arbitrary",),
        ),
    )(index_x, index_y, y_flat)


def _sc_dz(indices, index_x, index_y):
    mesh = plsc.VectorSubcoreMesh(
        core_axis_name="c", subcore_axis_name="s", num_cores=NC, num_subcores=NS
    )

    @functools.partial(
        pl.kernel,
        out_type=[
            jax.ShapeDtypeStruct((32 * PE,), jnp.float32),
            jax.ShapeDtypeStruct((64 * PE,), jnp.float32),
        ],
        mesh=mesh,
        scratch_types=[
            pltpu.VMEM((16,), jnp.int32),
            pltpu.VMEM((64,), jnp.int32),
            pltpu.VMEM((64,), jnp.int32),
            pltpu.VMEM((5 * SE,), jnp.float32),
            pltpu.SemaphoreType.DMA,
        ],
    )
    def k(ind_hbm, ix_hbm, iy_hbm, d_hbm, z_hbm, ind_v, ix_v, iy_v,
          tmpl_v, sem):
        cid = lax.axis_index("c")
        sid = lax.axis_index("s")
        wid = sid * NC + cid
        base = wid * SE

        pltpu.sync_copy(ind_hbm, ind_v.at[pl.ds(0, 8)])
        pltpu.sync_copy(ix_hbm, ix_v)
        pltpu.sync_copy(iy_hbm, iy_v)

        one = jnp.full((16,), 1.0, jnp.float32)
        zero = jnp.zeros((16,), jnp.float32)
        four = jnp.full((16,), 4.0, jnp.float32)

        def fill(kk):
            tmpl_v[pl.ds(0 * SE + kk * 16, 16)] = one
            tmpl_v[pl.ds(1 * SE + kk * 16, 16)] = zero
            tmpl_v[pl.ds(2 * SE + kk * 16, 16)] = zero
            tmpl_v[pl.ds(3 * SE + kk * 16, 16)] = four
            tmpl_v[pl.ds(4 * SE + kk * 16, 16)] = four

        pl.loop(0, SE // 16)(fill)

        @pl.when(wid == 0)
        def _rows():
            three = jnp.full((16,), 3.0, jnp.float32)

            def fix(kk):
                for r in (3, 5, 7, 9):
                    tmpl_v[pl.ds(r * W + kk * 16, 16)] = three

            pl.loop(0, W // 16)(fix)

        lane = lax.iota(jnp.int32, 16)

        @pl.when(wid == 0)
        def _corner():
            for q in range(4):
                hv = ix_v[pl.ds(q * 16, 16)]
                wv = iy_v[pl.ds(q * 16, 16)]
                for l in range(16):
                    h = hv[l]
                    w = wv[l]
                    for t in (1, 4):
                        off = t * SE + h * W
                        row = tmpl_v[pl.ds(off, 16)]
                        tmpl_v[pl.ds(off, 16)] = jnp.where(
                            lane == w, 1.0, row
                        )

        ind_vec = ind_v[pl.ds(0, 16)]
        copies = []
        for j in range(32):
            t = _tid(ind_vec[j // 4], j % 4)
            copies.append(
                pltpu.async_copy(
                    tmpl_v.at[pl.ds(t * SE, SE)],
                    d_hbm.at[pl.ds(j * PE + base, SE)],
                    sem,
                )
            )
        for i in range(64):
            q = i // 16
            bz = ix_v[pl.ds(q * 16, 16)][i % 16]
            cz = iy_v[pl.ds(q * 16, 16)][i % 16]
            t = _tid(bz, cz)
            copies.append(
                pltpu.async_copy(
                    tmpl_v.at[pl.ds(t * SE, SE)],
                    z_hbm.at[pl.ds(i * PE + base, SE)],
                    sem,
                )
            )
        for cp in copies:
            cp.wait()

    return k(indices, index_x, index_y)


@jax.jit
def kernel(x, y, indices, index_x, index_y):
    del x
    out = _tc_out(y.reshape(P, H, W), index_x, index_y)
    d, z = _sc_dz(indices, index_x, index_y)
    return (
        out.reshape(B, C, H, W),
        d.reshape(8, C, H, W),
        z.reshape(64, H, W),
    )
